# baseline (speedup 1.0000x reference)
import jax
import jax.numpy as jnp
from jax.experimental import pallas as pl
from jax.experimental.pallas import tpu as pltpu

_CHUNK = 256
_NBUF = 16
_UNROLL = 2


def _router_body(x_hbm, w_ref, o_ref, buf, sems):
    M = x_hbm.shape[0]
    nchunks = M // _CHUNK
    w = w_ref[...]

    def _copy(chunk_idx, slot):
        return pltpu.make_async_copy(
            x_hbm.at[pl.ds(chunk_idx * _CHUNK, _CHUNK), :],
            buf.at[slot],
            sems.at[slot],
        )

    for s in range(_NBUF):
        _copy(s, s).start()

    def _do_chunk(ci):
        slot = jax.lax.rem(ci, _NBUF)
        _copy(ci, slot).wait()
        x = buf[slot]
        logits = jax.lax.dot_general(
            x, w,
            dimension_numbers=(((1,), (1,)), ((), ())),
            preferred_element_type=jnp.float32,
        )
        m = jnp.max(logits, axis=-1, keepdims=True)
        e = jnp.exp(logits - m)
        o_ref[pl.ds(ci * _CHUNK, _CHUNK), :] = e / jnp.sum(e, axis=-1, keepdims=True)

        @pl.when(ci + _NBUF < nchunks)
        def _():
            _copy(ci + _NBUF, slot).start()

    def step(i, carry):
        base = i * _UNROLL
        for u in range(_UNROLL):
            _do_chunk(base + u)
        return carry

    jax.lax.fori_loop(0, nchunks // _UNROLL, step, 0)


def kernel(inputs, W):
    M, K = inputs.shape
    E = W.shape[0]
    return pl.pallas_call(
        _router_body,
        in_specs=[
            pl.BlockSpec(memory_space=pltpu.MemorySpace.HBM),
            pl.BlockSpec((E, K), lambda: (0, 0)),
        ],
        out_specs=pl.BlockSpec((M, E), lambda: (0, 0)),
        out_shape=jax.ShapeDtypeStruct((M, E), jnp.float32),
        scratch_shapes=[
            pltpu.VMEM((_NBUF, _CHUNK, K), jnp.float32),
            pltpu.SemaphoreType.DMA((_NBUF,)),
        ],
    )(inputs, W)
